# Initial kernel scaffold; baseline (speedup 1.0000x reference)
import jax
import jax.numpy as jnp
from jax.experimental import pallas as pl


def kernel(inputs):
    raise NotImplementedError("write your pallas kernel here")


# speedup vs baseline: 1.5580x; 1.5580x over previous
import jax
import jax.numpy as jnp
from jax import lax
from jax.experimental import pallas as pl
from jax.experimental.pallas import tpu as pltpu
from jax.experimental.pallas import tpu_sc as plsc

KWTA_K = 256
ROWS = 64
COLS = 8192
NUM_CORES = 2
NUM_SUBCORES = 16
NUM_WORKERS = NUM_CORES * NUM_SUBCORES
ROWS_PER_W = ROWS // NUM_WORKERS
LANES = 16
NVREG = COLS // LANES


def _kwta_body(in_hbm, out_hbm, x_v, key_v, out_v):
    wid = lax.axis_index("s") * NUM_CORES + lax.axis_index("c")
    base = wid * ROWS_PER_W
    pltpu.sync_copy(in_hbm.at[pl.ds(base, ROWS_PER_W)], x_v)

    ones = jnp.ones((LANES,), jnp.int32)
    zeros_i = jnp.zeros((LANES,), jnp.int32)
    zeros_f = jnp.zeros((LANES,), jnp.float32)

    for r in range(ROWS_PER_W):
        def key_body(i, carry):
            sl = pl.ds(i * LANES, LANES)
            bits = lax.bitcast_convert_type(x_v[r, sl], jnp.int32)
            flip = lax.shift_right_arithmetic(bits, 31) & jnp.int32(0x7FFFFFFF)
            key_v[r, sl] = bits ^ flip
            return carry

        lax.fori_loop(0, NVREG, key_body, jnp.int32(0))

        k_vec = jnp.full((LANES,), KWTA_K, jnp.int32)

        def bit_body(b, prefix_vec):
            bit_vec = lax.shift_left(ones, jnp.full((LANES,), 31 - b, jnp.int32))
            cand_vec = prefix_vec + bit_vec

            def cnt_body(i, acc):
                kv = key_v[r, pl.ds(i * LANES, LANES)]
                return acc + plsc.all_reduce_population_count(kv >= cand_vec)

            acc = lax.fori_loop(0, NVREG, cnt_body, zeros_i)
            return jnp.where(acc >= k_vec, cand_vec, prefix_vec)

        thr_vec = lax.fori_loop(
            0, 32, bit_body, jnp.full((LANES,), -2**31, jnp.int32))

        def mask_body(i, carry):
            sl = pl.ds(i * LANES, LANES)
            keep = key_v[r, sl] >= thr_vec
            out_v[r, sl] = jnp.where(keep, x_v[r, sl], zeros_f)
            return carry

        lax.fori_loop(0, NVREG, mask_body, jnp.int32(0))

    pltpu.sync_copy(out_v, out_hbm.at[pl.ds(base, ROWS_PER_W)])


def kernel(inputs):
    mesh = plsc.VectorSubcoreMesh(core_axis_name="c", subcore_axis_name="s")
    fn = pl.kernel(
        _kwta_body,
        mesh=mesh,
        out_type=jax.ShapeDtypeStruct((ROWS, COLS), jnp.float32),
        scratch_types=[
            pltpu.VMEM((ROWS_PER_W, COLS), jnp.float32),
            pltpu.VMEM((ROWS_PER_W, COLS), jnp.int32),
            pltpu.VMEM((ROWS_PER_W, COLS), jnp.float32),
        ],
        compiler_params=pltpu.CompilerParams(needs_layout_passes=False),
    )
    return fn(inputs)


# speedup vs baseline: 5.8434x; 3.7505x over previous
import jax
import jax.numpy as jnp
from jax import lax
from jax.experimental import pallas as pl
from jax.experimental.pallas import tpu as pltpu
from jax.experimental.pallas import tpu_sc as plsc

KWTA_K = 256
ROWS = 64
COLS = 8192
NUM_CORES = 2
NUM_SUBCORES = 16
NUM_WORKERS = NUM_CORES * NUM_SUBCORES
ROWS_PER_W = ROWS // NUM_WORKERS
LANES = 16
NVREG = COLS // LANES
UNROLL = 8

_popcount = plsc.all_reduce_population_count


def _kwta_body(in_hbm, out_hbm, x_v, key_v, out_v):
    wid = lax.axis_index("s") * NUM_CORES + lax.axis_index("c")
    base = wid * ROWS_PER_W
    pltpu.sync_copy(in_hbm.at[pl.ds(base, ROWS_PER_W)], x_v)

    ones = jnp.ones((LANES,), jnp.int32)
    zeros_i = jnp.zeros((LANES,), jnp.int32)
    zeros_f = jnp.zeros((LANES,), jnp.float32)
    k_vec = jnp.full((LANES,), KWTA_K, jnp.int32)
    low31 = jnp.full((LANES,), 0x7FFFFFFF, jnp.int32)
    R = ROWS_PER_W

    def key_body(i, accs):
        accs = list(accs)
        for j in range(UNROLL):
            sl = pl.ds((i * UNROLL + j) * LANES, LANES)
            for r in range(R):
                bits = lax.bitcast_convert_type(x_v[r, sl], jnp.int32)
                key = bits ^ (lax.shift_right_arithmetic(bits, 31) & low31)
                key_v[r, sl] = key
                accs[r] = accs[r] + _popcount(key >= zeros_i)
        return tuple(accs)

    accs = lax.fori_loop(0, NVREG // UNROLL, key_body, (zeros_i,) * R)
    int_min = jnp.full((LANES,), -2**31, jnp.int32)
    prefixes = tuple(
        jnp.where(acc >= k_vec, zeros_i, int_min) for acc in accs)

    def bit_body(b, prefixes):
        bit_vec = lax.shift_left(ones, jnp.full((LANES,), 30 - b, jnp.int32))
        cands = tuple(p + bit_vec for p in prefixes)

        def cnt_body(i, accs):
            accs = list(accs)
            for j in range(UNROLL):
                sl = pl.ds((i * UNROLL + j) * LANES, LANES)
                for r in range(R):
                    accs[r] = accs[r] + _popcount(key_v[r, sl] >= cands[r])
            return tuple(accs)

        accs = lax.fori_loop(0, NVREG // UNROLL, cnt_body, (zeros_i,) * R)
        return tuple(
            jnp.where(acc >= k_vec, cand, p)
            for acc, cand, p in zip(accs, cands, prefixes))

    thrs = lax.fori_loop(0, 31, bit_body, prefixes)

    def mask_body(i, carry):
        for j in range(UNROLL):
            sl = pl.ds((i * UNROLL + j) * LANES, LANES)
            for r in range(R):
                keep = key_v[r, sl] >= thrs[r]
                out_v[r, sl] = jnp.where(keep, x_v[r, sl], zeros_f)
        return carry

    lax.fori_loop(0, NVREG // UNROLL, mask_body, jnp.int32(0))

    pltpu.sync_copy(out_v, out_hbm.at[pl.ds(base, ROWS_PER_W)])


def kernel(inputs):
    mesh = plsc.VectorSubcoreMesh(core_axis_name="c", subcore_axis_name="s")
    fn = pl.kernel(
        _kwta_body,
        mesh=mesh,
        out_type=jax.ShapeDtypeStruct((ROWS, COLS), jnp.float32),
        scratch_types=[
            pltpu.VMEM((ROWS_PER_W, COLS), jnp.float32),
            pltpu.VMEM((ROWS_PER_W, COLS), jnp.int32),
            pltpu.VMEM((ROWS_PER_W, COLS), jnp.float32),
        ],
        compiler_params=pltpu.CompilerParams(needs_layout_passes=False),
    )
    return fn(inputs)
